# baseline (speedup 1.0000x reference)
import functools

import jax
import jax.numpy as jnp
from jax import lax
from jax.experimental import pallas as pl
from jax.experimental.pallas import tpu as pltpu
from jax.experimental.pallas import tpu_sc as plsc

NC = 2
NS = 16
CH = 80
NB = 2


def _sc_partials(x, src, dst):
    N, D = x.shape
    E = src.shape[0]
    NW = NC * NS
    epw = E // NW
    assert epw * NW == E and epw % CH == 0 and epw % 8 == 0
    n_chunks = epw // CH
    assert n_chunks >= NB
    rpt = (N // NS) // 8 * 8
    rem = N - NS * rpt
    assert rem % 8 == 0 and rem >= 0

    mesh = plsc.VectorSubcoreMesh(core_axis_name="c", subcore_axis_name="s")

    @functools.partial(
        pl.kernel,
        out_type=jax.ShapeDtypeStruct((NC * N, D), jnp.float32),
        mesh=mesh,
        scratch_types=[
            pltpu.VMEM((epw,), jnp.int32),
            pltpu.VMEM((n_chunks, CH), jnp.int32),
            [pltpu.VMEM((CH, D), jnp.float32) for _ in range(NB)],
            pltpu.VMEM_SHARED((N, D), jnp.float32),
            [pltpu.SemaphoreType.DMA for _ in range(NB)],
            [pltpu.SemaphoreType.DMA for _ in range(NB)],
        ],
    )
    def sc_kernel(x_hbm, src_hbm, dst_hbm, out_hbm, src_all, dst2d, rows,
                  acc, gsems, ssems):
        cid = lax.axis_index("c")
        sid = lax.axis_index("s")
        wid = sid * NC + cid
        ebase = wid * epw
        rbase = sid * rpt

        pltpu.sync_copy(src_hbm.at[pl.ds(ebase, epw)], src_all)
        pltpu.sync_copy(dst_hbm.at[wid], dst2d)

        pltpu.sync_copy(x_hbm.at[pl.ds(rbase, rpt)], acc.at[pl.ds(rbase, rpt)])
        if rem:
            @pl.when(sid == NS - 1)
            def _():
                pltpu.sync_copy(x_hbm.at[pl.ds(NS * rpt, rem)],
                                acc.at[pl.ds(NS * rpt, rem)])
        plsc.subcore_barrier()

        def drain(sem, buf):
            pltpu.make_async_copy(x_hbm.at[pl.ds(0, CH)], buf, sem).wait()

        def fire_gather(i, b):
            pltpu.async_copy(x_hbm.at[src_all.at[pl.ds(i * CH, CH)]],
                             rows[b], gsems[b])

        def fire_scatter(i, b):
            pltpu.async_copy(rows[b], acc.at[dst2d.at[i]], ssems[b], add=True)

        fire_gather(0, 0)

        def outer(g, carry):
            for b in range(NB):
                i = g * NB + b
                o = (b + 1) % NB
                drain(gsems[b], rows[b])
                fire_scatter(i, b)
                if b == 0:
                    @pl.when(i >= 1)
                    def _():
                        drain(ssems[o], rows[o])
                else:
                    drain(ssems[o], rows[o])
                fire_gather(i + 1, o)
            return carry

        n_main = ((n_chunks - 1) // NB) * NB
        lax.fori_loop(0, (n_chunks - 1) // NB, outer, 0)

        for r in range(n_main, n_chunks):
            b = r % NB
            o = (b + 1) % NB
            drain(gsems[b], rows[b])
            fire_scatter(r, b)
            if r >= 1:
                drain(ssems[o], rows[o])
            if r + 1 < n_chunks:
                fire_gather(r + 1, o)
        drain(ssems[(n_chunks - 1) % NB], rows[(n_chunks - 1) % NB])

        plsc.subcore_barrier()

        pltpu.sync_copy(acc.at[pl.ds(rbase, rpt)],
                        out_hbm.at[pl.ds(cid * N + rbase, rpt)])
        if rem:
            @pl.when(sid == NS - 1)
            def _():
                pltpu.sync_copy(acc.at[pl.ds(NS * rpt, rem)],
                                out_hbm.at[pl.ds(cid * N + NS * rpt, rem)])

    return sc_kernel(x, src, dst.reshape(NW, n_chunks, CH))


def _combine(partials, x):
    N, D = x.shape
    br = 2000
    grid = N // br
    assert br * grid == N

    def body(p0_ref, p1_ref, x_ref, o_ref):
        o_ref[...] = p0_ref[...] + p1_ref[...] - x_ref[...]

    return pl.pallas_call(
        body,
        grid=(grid,),
        in_specs=[
            pl.BlockSpec((br, D), lambda i: (i, 0)),
            pl.BlockSpec((br, D), lambda i: (i + grid, 0)),
            pl.BlockSpec((br, D), lambda i: (i, 0)),
        ],
        out_specs=pl.BlockSpec((br, D), lambda i: (i, 0)),
        out_shape=jax.ShapeDtypeStruct((N, D), jnp.float32),
    )(partials, partials, x)


def kernel(x, edge_index):
    src = edge_index[0]
    dst = edge_index[1]
    partials = _sc_partials(x, src, dst)
    return _combine(partials, x)


# speedup vs baseline: 1.2347x; 1.2347x over previous
import functools

import jax
import jax.numpy as jnp
from jax import lax
from jax.experimental import pallas as pl
from jax.experimental.pallas import tpu as pltpu
from jax.experimental.pallas import tpu_sc as plsc

NC = 2
NS = 16
CH = 80
NB = 2


def _sc_partials(x, src, dst):
    N, D = x.shape
    E = src.shape[0]
    NW = NC * NS
    epw = E // NW
    assert epw * NW == E and epw % CH == 0 and epw % 8 == 0
    n_chunks = epw // CH
    assert n_chunks >= NB
    rpt = (N // NS) // 8 * 8
    rem = N - NS * rpt
    assert rem % 8 == 0 and rem >= 0

    mesh = plsc.VectorSubcoreMesh(core_axis_name="c", subcore_axis_name="s")

    @functools.partial(
        pl.kernel,
        out_type=jax.ShapeDtypeStruct((NC * N, D), jnp.float32),
        mesh=mesh,
        scratch_types=[
            pltpu.VMEM((epw,), jnp.int32),
            pltpu.VMEM((n_chunks, CH), jnp.int32),
            [pltpu.VMEM((CH, D), jnp.float32) for _ in range(NB)],
            pltpu.VMEM_SHARED((N, D), jnp.float32),
            [pltpu.SemaphoreType.DMA for _ in range(NB)],
            [pltpu.SemaphoreType.DMA for _ in range(NB)],
        ],
    )
    def sc_kernel(x_hbm, src_hbm, dst_hbm, out_hbm, src_all, dst2d, rows,
                  acc, gsems, ssems):
        cid = lax.axis_index("c")
        sid = lax.axis_index("s")
        wid = sid * NC + cid
        ebase = wid * epw
        rbase = sid * rpt

        pltpu.sync_copy(src_hbm.at[pl.ds(ebase, epw)], src_all)
        pltpu.sync_copy(dst_hbm.at[wid], dst2d)

        pltpu.sync_copy(x_hbm.at[pl.ds(rbase, rpt)], acc.at[pl.ds(rbase, rpt)])
        if rem:
            @pl.when(sid == NS - 1)
            def _():
                pltpu.sync_copy(x_hbm.at[pl.ds(NS * rpt, rem)],
                                acc.at[pl.ds(NS * rpt, rem)])
        plsc.subcore_barrier()

        def drain(sem, buf):
            pltpu.make_async_copy(x_hbm.at[pl.ds(0, CH)], buf, sem).wait()

        def fire_gather(i, b):
            pltpu.async_copy(x_hbm.at[src_all.at[pl.ds(i * CH, CH)]],
                             rows[b], gsems[b])

        def fire_scatter(i, b):
            pltpu.async_copy(rows[b], acc.at[dst2d.at[i]], ssems[b], add=True)

        for b in range(NB):
            fire_gather(b, b)

        def outer(g, carry):
            for b in range(NB):
                i = g * NB + b
                drain(gsems[b], rows[b])
                pltpu.sync_copy(rows[b], acc.at[dst2d.at[i]], add=True)

                @pl.when(i + NB < n_chunks)
                def _():
                    fire_gather(i + NB, b)
            return carry

        lax.fori_loop(0, n_chunks // NB, outer, 0)

        for r in range((n_chunks // NB) * NB, n_chunks):
            b = r % NB
            drain(gsems[b], rows[b])
            pltpu.sync_copy(rows[b], acc.at[dst2d.at[r]], add=True)

        plsc.subcore_barrier()

        pltpu.sync_copy(acc.at[pl.ds(rbase, rpt)],
                        out_hbm.at[pl.ds(cid * N + rbase, rpt)])
        if rem:
            @pl.when(sid == NS - 1)
            def _():
                pltpu.sync_copy(acc.at[pl.ds(NS * rpt, rem)],
                                out_hbm.at[pl.ds(cid * N + NS * rpt, rem)])

    return sc_kernel(x, src, dst.reshape(NW, n_chunks, CH))


def _combine(partials, x):
    N, D = x.shape
    br = 2000
    grid = N // br
    assert br * grid == N

    def body(p0_ref, p1_ref, x_ref, o_ref):
        o_ref[...] = p0_ref[...] + p1_ref[...] - x_ref[...]

    return pl.pallas_call(
        body,
        grid=(grid,),
        in_specs=[
            pl.BlockSpec((br, D), lambda i: (i, 0)),
            pl.BlockSpec((br, D), lambda i: (i + grid, 0)),
            pl.BlockSpec((br, D), lambda i: (i, 0)),
        ],
        out_specs=pl.BlockSpec((br, D), lambda i: (i, 0)),
        out_shape=jax.ShapeDtypeStruct((N, D), jnp.float32),
    )(partials, partials, x)


def kernel(x, edge_index):
    src = edge_index[0]
    dst = edge_index[1]
    partials = _sc_partials(x, src, dst)
    return _combine(partials, x)


# speedup vs baseline: 1.3130x; 1.0634x over previous
import functools

import jax
import jax.numpy as jnp
from jax import lax
from jax.experimental import pallas as pl
from jax.experimental.pallas import tpu as pltpu
from jax.experimental.pallas import tpu_sc as plsc

NC = 2
NS = 16
CH = 128
NB = 2
QR = 4
UN = 4


def _sc_partials(x, src, dst_main, dst_tail):
    N, D = x.shape
    E = src.shape[0]
    NW = NC * NS
    epw = E // NW
    assert epw * NW == E and epw % 8 == 0
    n_chunks = epw // CH
    tail_e = epw - n_chunks * CH
    assert tail_e % 8 == 0
    assert n_chunks > QR
    rpt = (N // NS) // 8 * 8
    rem = N - NS * rpt
    assert rem % 8 == 0 and rem >= 0

    mesh = plsc.VectorSubcoreMesh(core_axis_name="c", subcore_axis_name="s")

    @functools.partial(
        pl.kernel,
        out_type=jax.ShapeDtypeStruct((NC * N, D), jnp.float32),
        mesh=mesh,
        scratch_types=[
            [pltpu.VMEM((CH,), jnp.int32) for _ in range(QR)],
            pltpu.VMEM((max(tail_e, 8),), jnp.int32),
            pltpu.VMEM((max(tail_e, 8),), jnp.int32),
            pltpu.VMEM((n_chunks, CH), jnp.int32),
            pltpu.VMEM((NB * CH, D), jnp.float32),
            pltpu.VMEM_SHARED((N, D), jnp.float32),
            [pltpu.SemaphoreType.DMA for _ in range(NB)],
            [pltpu.SemaphoreType.DMA for _ in range(QR)],
        ],
    )
    def sc_kernel(x_hbm, src_hbm, dstm_hbm, dstt_hbm, out_hbm,
                  sidx, sidx_t, didx_t, dst2d, rows_all, acc, gsems, isems):
        rows = [rows_all.at[pl.ds(b * CH, CH)] for b in range(NB)]
        cid = lax.axis_index("c")
        sid = lax.axis_index("s")
        wid = sid * NC + cid
        ebase = wid * epw
        rbase = sid * rpt

        pltpu.sync_copy(dstm_hbm.at[wid], dst2d)

        pltpu.sync_copy(x_hbm.at[pl.ds(rbase, rpt)], acc.at[pl.ds(rbase, rpt)])
        if rem:
            @pl.when(sid == NS - 1)
            def _():
                pltpu.sync_copy(x_hbm.at[pl.ds(NS * rpt, rem)],
                                acc.at[pl.ds(NS * rpt, rem)])
        plsc.subcore_barrier()

        def drain(sem, buf_ref):
            pltpu.make_async_copy(x_hbm.at[pl.ds(0, CH)], buf_ref, sem).wait()

        def drain_idx(q):
            pltpu.make_async_copy(src_hbm.at[pl.ds(0, CH)], sidx[q],
                                  isems[q]).wait()

        def fire_idx(j, q):
            pltpu.async_copy(src_hbm.at[pl.ds(ebase + j * CH, CH)], sidx[q],
                             isems[q])

        def fire_gather(j, b, q):
            pltpu.async_copy(x_hbm.at[sidx[q]], rows[b], gsems[b])

        for q in range(QR):
            fire_idx(q, q)
        for b in range(NB):
            drain_idx(b)
            fire_gather(b, b, b)

        def step(i, b, q, do_fidx, do_fg):
            drain(gsems[b], rows[b])
            pltpu.sync_copy(rows[b], acc.at[dst2d.at[i]], add=True)
            if do_fidx:
                fire_idx(i + QR, q)
            if do_fg:
                drain_idx((q + NB) % QR)
                fire_gather(i + NB, b, (q + NB) % QR)

        n_body = n_chunks - QR
        n_grp = n_body // UN
        n_peeled = n_grp * UN

        def outer(g, carry):
            for u in range(UN):
                i = g * UN + u
                step(i, u % NB, u % QR, True, True)
            return carry

        lax.fori_loop(0, n_grp, outer, 0)

        for i in range(n_peeled, n_chunks):
            step(i, i % NB, i % QR, i + QR < n_chunks, i + NB < n_chunks)

        if tail_e:
            tb = ebase + n_chunks * CH
            pltpu.sync_copy(src_hbm.at[pl.ds(tb, tail_e)],
                            sidx_t.at[pl.ds(0, tail_e)])
            pltpu.sync_copy(dstt_hbm.at[pl.ds(wid * tail_e, tail_e)],
                            didx_t.at[pl.ds(0, tail_e)])
            pltpu.async_copy(x_hbm.at[sidx_t.at[pl.ds(0, tail_e)]],
                             rows_all.at[pl.ds(0, tail_e)], gsems[0]).wait()
            pltpu.sync_copy(rows_all.at[pl.ds(0, tail_e)],
                            acc.at[didx_t.at[pl.ds(0, tail_e)]], add=True)

        plsc.subcore_barrier()

        pltpu.sync_copy(acc.at[pl.ds(rbase, rpt)],
                        out_hbm.at[pl.ds(cid * N + rbase, rpt)])
        if rem:
            @pl.when(sid == NS - 1)
            def _():
                pltpu.sync_copy(acc.at[pl.ds(NS * rpt, rem)],
                                out_hbm.at[pl.ds(cid * N + NS * rpt, rem)])

    return sc_kernel(x, src, dst_main, dst_tail)


def _combine(partials, x):
    N, D = x.shape
    br = 2000
    grid = N // br
    assert br * grid == N

    def body(p0_ref, p1_ref, x_ref, o_ref):
        o_ref[...] = p0_ref[...] + p1_ref[...] - x_ref[...]

    return pl.pallas_call(
        body,
        grid=(grid,),
        in_specs=[
            pl.BlockSpec((br, D), lambda i: (i, 0)),
            pl.BlockSpec((br, D), lambda i: (i + grid, 0)),
            pl.BlockSpec((br, D), lambda i: (i, 0)),
        ],
        out_specs=pl.BlockSpec((br, D), lambda i: (i, 0)),
        out_shape=jax.ShapeDtypeStruct((N, D), jnp.float32),
    )(partials, partials, x)


def kernel(x, edge_index):
    src = edge_index[0]
    dst = edge_index[1]
    NW = NC * NS
    E = src.shape[0]
    epw = E // NW
    n_chunks = epw // CH
    tail_e = epw - n_chunks * CH
    dst_w = dst.reshape(NW, epw)
    dst_main = dst_w[:, :n_chunks * CH].reshape(NW, n_chunks, CH)
    dst_tail = dst_w[:, n_chunks * CH:].reshape(-1)
    partials = _sc_partials(x, src, dst_main, dst_tail)
    return _combine(partials, x)


# speedup vs baseline: 1.4347x; 1.0927x over previous
import functools

import jax
import jax.numpy as jnp
from jax import lax
from jax.experimental import pallas as pl
from jax.experimental.pallas import tpu as pltpu
from jax.experimental.pallas import tpu_sc as plsc

NC = 2
NS = 16
CH = 128
NB = 2
QR = 4
UN = 4


def _sc_partials(x, eidx3):
    N, D = x.shape
    nchk = eidx3.shape[0] // 2
    NW = NC * NS
    NL = nchk // NW
    extra = nchk - NL * NW
    assert extra < NW and NL > QR
    rpt = (N // NS) // 8 * 8
    rem = N - NS * rpt
    assert rem % 8 == 0 and rem >= 0

    mesh = plsc.VectorSubcoreMesh(core_axis_name="c", subcore_axis_name="s")

    @functools.partial(
        pl.kernel,
        out_type=jax.ShapeDtypeStruct((NC * N, D), jnp.float32),
        mesh=mesh,
        scratch_types=[
            [pltpu.VMEM((1, CH), jnp.int32) for _ in range(QR)],
            pltpu.VMEM((NL, 1, CH), jnp.int32),
            pltpu.VMEM((NB * CH, D), jnp.float32),
            pltpu.VMEM_SHARED((N, D), jnp.float32),
            [pltpu.SemaphoreType.DMA for _ in range(NB)],
            [pltpu.SemaphoreType.DMA for _ in range(QR)],
            pltpu.SemaphoreType.DMA,
        ],
    )
    def sc_kernel(x_hbm, eidx_hbm, out_hbm,
                  sidx, dst2d, rows_all, acc, gsems, isems, dsem):
        rows = [rows_all.at[pl.ds(b * CH, CH)] for b in range(NB)]
        cid = lax.axis_index("c")
        sid = lax.axis_index("s")
        wid = sid * NC + cid
        cbase = wid * NL
        rbase = sid * rpt

        def fill(i, c):
            pltpu.async_copy(eidx_hbm.at[nchk + cbase + i], dst2d.at[i], dsem)
            return c

        lax.fori_loop(0, NL, fill, 0)

        pltpu.sync_copy(x_hbm.at[pl.ds(rbase, rpt)], acc.at[pl.ds(rbase, rpt)])
        if rem:
            @pl.when(sid == NS - 1)
            def _():
                pltpu.sync_copy(x_hbm.at[pl.ds(NS * rpt, rem)],
                                acc.at[pl.ds(NS * rpt, rem)])
        pltpu.make_async_copy(eidx_hbm.at[pl.ds(0, NL)], dst2d, dsem).wait()
        plsc.subcore_barrier()

        def drain(sem, buf_ref):
            pltpu.make_async_copy(x_hbm.at[pl.ds(0, CH)], buf_ref, sem).wait()

        def drain_idx(q):
            pltpu.make_async_copy(eidx_hbm.at[0], sidx[q], isems[q]).wait()

        def fire_idx(j, q):
            pltpu.async_copy(eidx_hbm.at[cbase + j], sidx[q], isems[q])

        def fire_gather(b, q):
            pltpu.async_copy(x_hbm.at[sidx[q].at[0]], rows[b], gsems[b])

        for q in range(QR):
            fire_idx(q, q)
        for b in range(NB):
            drain_idx(b)
            fire_gather(b, b)

        def step(i, b, q, do_fidx, do_fg):
            drain(gsems[b], rows[b])
            pltpu.sync_copy(rows[b], acc.at[dst2d.at[i, 0]], add=True)
            if do_fidx:
                fire_idx(i + QR, q)
            if do_fg:
                drain_idx((q + NB) % QR)
                fire_gather(b, (q + NB) % QR)

        n_grp = (NL - QR) // UN
        n_peeled = n_grp * UN

        def outer(g, carry):
            for u in range(UN):
                step(g * UN + u, u % NB, u % QR, True, True)
            return carry

        lax.fori_loop(0, n_grp, outer, 0)

        for i in range(n_peeled, NL):
            step(i, i % NB, i % QR, i + QR < NL, i + NB < NL)

        if extra:
            @pl.when(wid < extra)
            def _():
                jj = nchk - extra + wid
                pltpu.sync_copy(eidx_hbm.at[jj], sidx[0])
                pltpu.sync_copy(eidx_hbm.at[nchk + jj], sidx[1])
                pltpu.async_copy(x_hbm.at[sidx[0].at[0]], rows[0],
                                 gsems[0]).wait()
                pltpu.sync_copy(rows[0], acc.at[sidx[1].at[0]], add=True)

        plsc.subcore_barrier()

        pltpu.sync_copy(acc.at[pl.ds(rbase, rpt)],
                        out_hbm.at[pl.ds(cid * N + rbase, rpt)])
        if rem:
            @pl.when(sid == NS - 1)
            def _():
                pltpu.sync_copy(acc.at[pl.ds(NS * rpt, rem)],
                                out_hbm.at[pl.ds(cid * N + NS * rpt, rem)])

    return sc_kernel(x, eidx3)


def _combine(partials, x):
    N, D = x.shape
    br = 2000
    grid = N // br
    assert br * grid == N

    def body(p0_ref, p1_ref, x_ref, o_ref):
        o_ref[...] = p0_ref[...] + p1_ref[...] - x_ref[...]

    return pl.pallas_call(
        body,
        grid=(grid,),
        in_specs=[
            pl.BlockSpec((br, D), lambda i: (i, 0)),
            pl.BlockSpec((br, D), lambda i: (i + grid, 0)),
            pl.BlockSpec((br, D), lambda i: (i, 0)),
        ],
        out_specs=pl.BlockSpec((br, D), lambda i: (i, 0)),
        out_shape=jax.ShapeDtypeStruct((N, D), jnp.float32),
    )(partials, partials, x)


def kernel(x, edge_index):
    E = edge_index.shape[1]
    assert E % CH == 0
    eidx3 = edge_index.reshape(2 * (E // CH), 1, CH)
    partials = _sc_partials(x, eidx3)
    return _combine(partials, x)


# speedup vs baseline: 1.4926x; 1.0403x over previous
import functools

import jax
import jax.numpy as jnp
from jax import lax
from jax.experimental import pallas as pl
from jax.experimental.pallas import tpu as pltpu
from jax.experimental.pallas import tpu_sc as plsc

NC = 2
NS = 16
CH = 128
NB = 2
QR = 4
UN = 4


def _sc_partials(x, eidx, dst3):
    N, D = x.shape
    nchk = dst3.shape[0]
    NW = NC * NS
    NL = nchk // NW
    extra = nchk - NL * NW
    assert extra < NW and NL > QR
    rpt = (N // NS) // 8 * 8
    rem = N - NS * rpt
    assert rem % 8 == 0 and rem >= 0

    mesh = plsc.VectorSubcoreMesh(core_axis_name="c", subcore_axis_name="s")

    @functools.partial(
        pl.kernel,
        out_type=jax.ShapeDtypeStruct((NC * N, D), jnp.float32),
        mesh=mesh,
        scratch_types=[
            [pltpu.VMEM((1, CH), jnp.int32) for _ in range(QR)],
            pltpu.VMEM((NL, 1, CH), jnp.int32),
            pltpu.VMEM((NB * CH, D), jnp.float32),
            pltpu.VMEM_SHARED((N, D), jnp.float32),
            [pltpu.SemaphoreType.DMA for _ in range(NB)],
            [pltpu.SemaphoreType.DMA for _ in range(QR)],
            pltpu.SemaphoreType.DMA,
            pltpu.SemaphoreType.DMA,
        ],
    )
    def sc_kernel(x_hbm, eidx_hbm, dst3_hbm, out_hbm,
                  sidx, dst2d, rows_all, acc, gsems, isems, dsem, nsem):
        rows = [rows_all.at[pl.ds(b * CH, CH)] for b in range(NB)]
        cid = lax.axis_index("c")
        sid = lax.axis_index("s")
        wid = sid * NC + cid
        cbase = wid * NL
        rbase = sid * rpt

        def fill(i, c):
            pltpu.async_copy(dst3_hbm.at[cbase + i], dst2d.at[i], dsem)
            return c

        lax.fori_loop(0, NL, fill, 0)

        pltpu.async_copy(x_hbm.at[pl.ds(rbase, rpt)],
                         acc.at[pl.ds(rbase, rpt)], nsem)
        if rem:
            @pl.when(sid == NS - 1)
            def _():
                pltpu.async_copy(x_hbm.at[pl.ds(NS * rpt, rem)],
                                 acc.at[pl.ds(NS * rpt, rem)], nsem)

        def drain(sem, buf_ref):
            pltpu.make_async_copy(x_hbm.at[pl.ds(0, CH)], buf_ref, sem).wait()

        def drain_idx(q):
            pltpu.make_async_copy(dst3_hbm.at[0], sidx[q], isems[q]).wait()

        def fire_idx(j, q):
            pltpu.async_copy(eidx_hbm.at[0, pl.ds((cbase + j) * CH, CH)],
                             sidx[q].at[0], isems[q])

        def fire_gather(b, q):
            pltpu.async_copy(x_hbm.at[sidx[q].at[0]], rows[b], gsems[b])

        for q in range(QR):
            fire_idx(q, q)
        for b in range(NB):
            drain_idx(b)
            fire_gather(b, b)

        pltpu.make_async_copy(x_hbm.at[pl.ds(0, rpt)],
                              acc.at[pl.ds(0, rpt)], nsem).wait()
        if rem:
            @pl.when(sid == NS - 1)
            def _():
                pltpu.make_async_copy(x_hbm.at[pl.ds(0, rem)],
                                      acc.at[pl.ds(0, rem)], nsem).wait()
        pltpu.make_async_copy(dst3_hbm.at[pl.ds(0, NL)], dst2d, dsem).wait()
        plsc.subcore_barrier()

        def step(i, b, q, do_fidx, do_fg):
            drain(gsems[b], rows[b])
            pltpu.sync_copy(rows[b], acc.at[dst2d.at[i, 0]], add=True)
            if do_fidx:
                fire_idx(i + QR, q)
            if do_fg:
                drain_idx((q + NB) % QR)
                fire_gather(b, (q + NB) % QR)

        n_grp = (NL - QR) // UN
        n_peeled = n_grp * UN

        def outer(g, carry):
            for u in range(UN):
                step(g * UN + u, u % NB, u % QR, True, True)
            return carry

        lax.fori_loop(0, n_grp, outer, 0)

        for i in range(n_peeled, NL):
            step(i, i % NB, i % QR, i + QR < NL, i + NB < NL)

        if extra:
            @pl.when(wid < extra)
            def _():
                jj = nchk - extra + wid
                pltpu.sync_copy(eidx_hbm.at[0, pl.ds(jj * CH, CH)],
                                sidx[0].at[0])
                pltpu.sync_copy(dst3_hbm.at[jj], sidx[1])
                pltpu.async_copy(x_hbm.at[sidx[0].at[0]], rows[0],
                                 gsems[0]).wait()
                pltpu.sync_copy(rows[0], acc.at[sidx[1].at[0]], add=True)

        plsc.subcore_barrier()

        pltpu.sync_copy(acc.at[pl.ds(rbase, rpt)],
                        out_hbm.at[pl.ds(cid * N + rbase, rpt)])
        if rem:
            @pl.when(sid == NS - 1)
            def _():
                pltpu.sync_copy(acc.at[pl.ds(NS * rpt, rem)],
                                out_hbm.at[pl.ds(cid * N + NS * rpt, rem)])

    return sc_kernel(x, eidx, dst3)


def _combine(partials, x):
    N, D = x.shape
    br = 2000
    grid = N // br
    assert br * grid == N

    def body(p0_ref, p1_ref, x_ref, o_ref):
        o_ref[...] = p0_ref[...] + p1_ref[...] - x_ref[...]

    return pl.pallas_call(
        body,
        grid=(grid,),
        in_specs=[
            pl.BlockSpec((br, D), lambda i: (i, 0)),
            pl.BlockSpec((br, D), lambda i: (i + grid, 0)),
            pl.BlockSpec((br, D), lambda i: (i, 0)),
        ],
        out_specs=pl.BlockSpec((br, D), lambda i: (i, 0)),
        out_shape=jax.ShapeDtypeStruct((N, D), jnp.float32),
    )(partials, partials, x)


def kernel(x, edge_index):
    E = edge_index.shape[1]
    assert E % CH == 0
    dst3 = edge_index[1].reshape(E // CH, 1, CH)
    partials = _sc_partials(x, edge_index, dst3)
    return _combine(partials, x)


# speedup vs baseline: 1.5041x; 1.0077x over previous
import functools

import jax
import jax.numpy as jnp
from jax import lax
from jax.experimental import pallas as pl
from jax.experimental.pallas import tpu as pltpu
from jax.experimental.pallas import tpu_sc as plsc

NC = 2
NS = 16
CH = 128
NB = 2
QR = 4
UN = 4


def _sc_partials(x, eidx):
    N, D = x.shape
    nchk = eidx.shape[1] // CH
    NW = NC * NS
    NL = nchk // NW
    extra = nchk - NL * NW
    assert extra < NW and NL > QR
    rpt = (N // NS) // 8 * 8
    rem = N - NS * rpt
    assert rem % 8 == 0 and rem >= 0

    mesh = plsc.VectorSubcoreMesh(core_axis_name="c", subcore_axis_name="s")

    @functools.partial(
        pl.kernel,
        out_type=jax.ShapeDtypeStruct((NC * N, D), jnp.float32),
        mesh=mesh,
        scratch_types=[
            [pltpu.VMEM((2, CH), jnp.int32) for _ in range(QR)],
            pltpu.VMEM((NB * CH, D), jnp.float32),
            pltpu.VMEM_SHARED((N, D), jnp.float32),
            [pltpu.SemaphoreType.DMA for _ in range(NB)],
            [pltpu.SemaphoreType.DMA for _ in range(QR)],
            pltpu.SemaphoreType.DMA,
        ],
    )
    def sc_kernel(x_hbm, eidx_hbm, out_hbm,
                  sidx, rows_all, acc, gsems, isems, nsem):
        rows = [rows_all.at[pl.ds(b * CH, CH)] for b in range(NB)]
        cid = lax.axis_index("c")
        sid = lax.axis_index("s")
        wid = sid * NC + cid
        cbase = wid * NL
        rbase = sid * rpt

        pltpu.async_copy(x_hbm.at[pl.ds(rbase, rpt)],
                         acc.at[pl.ds(rbase, rpt)], nsem)
        if rem:
            @pl.when(sid == NS - 1)
            def _():
                pltpu.async_copy(x_hbm.at[pl.ds(NS * rpt, rem)],
                                 acc.at[pl.ds(NS * rpt, rem)], nsem)

        def drain(sem, buf_ref):
            pltpu.make_async_copy(x_hbm.at[pl.ds(0, CH)], buf_ref, sem).wait()

        def drain_idx(q):
            pltpu.make_async_copy(eidx_hbm.at[pl.ds(0, 2), pl.ds(0, CH)],
                                  sidx[q], isems[q]).wait()

        def fire_idx(j, q):
            pltpu.async_copy(
                eidx_hbm.at[pl.ds(0, 2), pl.ds((cbase + j) * CH, CH)],
                sidx[q], isems[q])

        def fire_gather(b, q):
            pltpu.async_copy(x_hbm.at[sidx[q].at[0]], rows[b], gsems[b])

        for q in range(QR):
            fire_idx(q, q)
        for b in range(NB):
            drain_idx(b)
            fire_gather(b, b)

        pltpu.make_async_copy(x_hbm.at[pl.ds(0, rpt)],
                              acc.at[pl.ds(0, rpt)], nsem).wait()
        if rem:
            @pl.when(sid == NS - 1)
            def _():
                pltpu.make_async_copy(x_hbm.at[pl.ds(0, rem)],
                                      acc.at[pl.ds(0, rem)], nsem).wait()
        plsc.subcore_barrier()

        def step(i, b, q, do_fidx, do_fg):
            drain(gsems[b], rows[b])
            pltpu.sync_copy(rows[b], acc.at[sidx[q].at[1]], add=True)
            if do_fidx:
                fire_idx(i + QR, q)
            if do_fg:
                drain_idx((q + NB) % QR)
                fire_gather(b, (q + NB) % QR)

        n_grp = (NL - QR) // UN
        n_peeled = n_grp * UN

        def outer(g, carry):
            for u in range(UN):
                step(g * UN + u, u % NB, u % QR, True, True)
            return carry

        lax.fori_loop(0, n_grp, outer, 0)

        for i in range(n_peeled, NL):
            step(i, i % NB, i % QR, i + QR < NL, i + NB < NL)

        if extra:
            @pl.when(wid < extra)
            def _():
                jj = nchk - extra + wid
                pltpu.sync_copy(
                    eidx_hbm.at[pl.ds(0, 2), pl.ds(jj * CH, CH)], sidx[0])
                pltpu.async_copy(x_hbm.at[sidx[0].at[0]], rows[0],
                                 gsems[0]).wait()
                pltpu.sync_copy(rows[0], acc.at[sidx[0].at[1]], add=True)

        plsc.subcore_barrier()

        pltpu.sync_copy(acc.at[pl.ds(rbase, rpt)],
                        out_hbm.at[pl.ds(cid * N + rbase, rpt)])
        if rem:
            @pl.when(sid == NS - 1)
            def _():
                pltpu.sync_copy(acc.at[pl.ds(NS * rpt, rem)],
                                out_hbm.at[pl.ds(cid * N + NS * rpt, rem)])

    return sc_kernel(x, eidx)


def _combine(partials, x):
    N, D = x.shape
    br = 2000
    grid = N // br
    assert br * grid == N

    def body(p0_ref, p1_ref, x_ref, o_ref):
        o_ref[...] = p0_ref[...] + p1_ref[...] - x_ref[...]

    return pl.pallas_call(
        body,
        grid=(grid,),
        in_specs=[
            pl.BlockSpec((br, D), lambda i: (i, 0)),
            pl.BlockSpec((br, D), lambda i: (i + grid, 0)),
            pl.BlockSpec((br, D), lambda i: (i, 0)),
        ],
        out_specs=pl.BlockSpec((br, D), lambda i: (i, 0)),
        out_shape=jax.ShapeDtypeStruct((N, D), jnp.float32),
    )(partials, partials, x)


def kernel(x, edge_index):
    E = edge_index.shape[1]
    assert E % CH == 0
    partials = _sc_partials(x, edge_index)
    return _combine(partials, x)


# speedup vs baseline: 1.6660x; 1.1077x over previous
import functools

import jax
import jax.numpy as jnp
from jax import lax
from jax.experimental import pallas as pl
from jax.experimental.pallas import tpu as pltpu
from jax.experimental.pallas import tpu_sc as plsc

NC = 2
NS = 16
CH = 128
NB = 3
QR = 4
UN = 12


def _sc_partials(x, eidx):
    N, D = x.shape
    nchk = eidx.shape[1] // CH
    NW = NC * NS
    NL = nchk // NW
    extra = nchk - NL * NW
    assert extra < NW and NL > QR
    rpt = (N // NS) // 8 * 8
    rem = N - NS * rpt
    assert rem % 8 == 0 and rem >= 0

    mesh = plsc.VectorSubcoreMesh(core_axis_name="c", subcore_axis_name="s")

    @functools.partial(
        pl.kernel,
        out_type=jax.ShapeDtypeStruct((NC * N, D), jnp.float32),
        mesh=mesh,
        scratch_types=[
            [pltpu.VMEM((2, CH), jnp.int32) for _ in range(QR)],
            pltpu.VMEM((NB * CH, D), jnp.float32),
            pltpu.VMEM_SHARED((N, D), jnp.float32),
            [pltpu.SemaphoreType.DMA for _ in range(NB)],
            [pltpu.SemaphoreType.DMA for _ in range(QR)],
            pltpu.SemaphoreType.DMA,
        ],
    )
    def sc_kernel(x_hbm, eidx_hbm, out_hbm,
                  sidx, rows_all, acc, gsems, isems, nsem):
        rows = [rows_all.at[pl.ds(b * CH, CH)] for b in range(NB)]
        cid = lax.axis_index("c")
        sid = lax.axis_index("s")
        wid = sid * NC + cid
        cbase = wid * NL
        rbase = sid * rpt

        pltpu.async_copy(x_hbm.at[pl.ds(rbase, rpt)],
                         acc.at[pl.ds(rbase, rpt)], nsem)
        if rem:
            @pl.when(sid == NS - 1)
            def _():
                pltpu.async_copy(x_hbm.at[pl.ds(NS * rpt, rem)],
                                 acc.at[pl.ds(NS * rpt, rem)], nsem)

        def drain(sem, buf_ref):
            pltpu.make_async_copy(x_hbm.at[pl.ds(0, CH)], buf_ref, sem).wait()

        def drain_idx(q):
            pltpu.make_async_copy(eidx_hbm.at[pl.ds(0, 2), pl.ds(0, CH)],
                                  sidx[q], isems[q]).wait()

        def fire_idx(j, q):
            pltpu.async_copy(
                eidx_hbm.at[pl.ds(0, 2), pl.ds((cbase + j) * CH, CH)],
                sidx[q], isems[q])

        def fire_gather(b, q):
            pltpu.async_copy(x_hbm.at[sidx[q].at[0]], rows[b], gsems[b])

        for q in range(QR):
            fire_idx(q, q)
        for b in range(NB):
            drain_idx(b)
            fire_gather(b, b)

        pltpu.make_async_copy(x_hbm.at[pl.ds(0, rpt)],
                              acc.at[pl.ds(0, rpt)], nsem).wait()
        if rem:
            @pl.when(sid == NS - 1)
            def _():
                pltpu.make_async_copy(x_hbm.at[pl.ds(0, rem)],
                                      acc.at[pl.ds(0, rem)], nsem).wait()
        plsc.subcore_barrier()

        def step(i, b, q, do_fidx, do_fg):
            drain(gsems[b], rows[b])
            pltpu.sync_copy(rows[b], acc.at[sidx[q].at[1]], add=True)
            if do_fidx:
                fire_idx(i + QR, q)
            if do_fg:
                drain_idx((q + NB) % QR)
                fire_gather(b, (q + NB) % QR)

        n_grp = (NL - QR) // UN
        n_peeled = n_grp * UN

        def outer(g, carry):
            for u in range(UN):
                step(g * UN + u, u % NB, u % QR, True, True)
            return carry

        lax.fori_loop(0, n_grp, outer, 0)

        for i in range(n_peeled, NL):
            step(i, i % NB, i % QR, i + QR < NL, i + NB < NL)

        if extra:
            @pl.when(wid < extra)
            def _():
                jj = nchk - extra + wid
                pltpu.sync_copy(
                    eidx_hbm.at[pl.ds(0, 2), pl.ds(jj * CH, CH)], sidx[0])
                pltpu.async_copy(x_hbm.at[sidx[0].at[0]], rows[0],
                                 gsems[0]).wait()
                pltpu.sync_copy(rows[0], acc.at[sidx[0].at[1]], add=True)

        plsc.subcore_barrier()

        pltpu.sync_copy(acc.at[pl.ds(rbase, rpt)],
                        out_hbm.at[pl.ds(cid * N + rbase, rpt)])
        if rem:
            @pl.when(sid == NS - 1)
            def _():
                pltpu.sync_copy(acc.at[pl.ds(NS * rpt, rem)],
                                out_hbm.at[pl.ds(cid * N + NS * rpt, rem)])

    return sc_kernel(x, eidx)


def _combine(partials, x):
    N, D = x.shape
    br = 2000
    grid = N // br
    assert br * grid == N

    def body(p0_ref, p1_ref, x_ref, o_ref):
        o_ref[...] = p0_ref[...] + p1_ref[...] - x_ref[...]

    return pl.pallas_call(
        body,
        grid=(grid,),
        in_specs=[
            pl.BlockSpec((br, D), lambda i: (i, 0)),
            pl.BlockSpec((br, D), lambda i: (i + grid, 0)),
            pl.BlockSpec((br, D), lambda i: (i, 0)),
        ],
        out_specs=pl.BlockSpec((br, D), lambda i: (i, 0)),
        out_shape=jax.ShapeDtypeStruct((N, D), jnp.float32),
    )(partials, partials, x)


def kernel(x, edge_index):
    E = edge_index.shape[1]
    assert E % CH == 0
    partials = _sc_partials(x, edge_index)
    return _combine(partials, x)


# speedup vs baseline: 1.6671x; 1.0007x over previous
import functools

import jax
import jax.numpy as jnp
from jax import lax
from jax.experimental import pallas as pl
from jax.experimental.pallas import tpu as pltpu
from jax.experimental.pallas import tpu_sc as plsc

NC = 2
NS = 16
CH = 128
NB = 3
QR = 6
UN = 6


def _sc_partials(x, eidx):
    N, D = x.shape
    nchk = eidx.shape[1] // CH
    NW = NC * NS
    NL = nchk // NW
    extra = nchk - NL * NW
    assert extra < NW and NL > QR
    rpt = (N // NS) // 8 * 8
    rem = N - NS * rpt
    assert rem % 8 == 0 and rem >= 0

    mesh = plsc.VectorSubcoreMesh(core_axis_name="c", subcore_axis_name="s")

    @functools.partial(
        pl.kernel,
        out_type=jax.ShapeDtypeStruct((NC * N, D), jnp.float32),
        mesh=mesh,
        scratch_types=[
            [pltpu.VMEM((2, CH), jnp.int32) for _ in range(QR)],
            pltpu.VMEM((NB * CH, D), jnp.float32),
            pltpu.VMEM_SHARED((N, D), jnp.float32),
            [pltpu.SemaphoreType.DMA for _ in range(NB)],
            [pltpu.SemaphoreType.DMA for _ in range(QR)],
            pltpu.SemaphoreType.DMA,
        ],
    )
    def sc_kernel(x_hbm, eidx_hbm, out_hbm,
                  sidx, rows_all, acc, gsems, isems, nsem):
        rows = [rows_all.at[pl.ds(b * CH, CH)] for b in range(NB)]
        cid = lax.axis_index("c")
        sid = lax.axis_index("s")
        wid = sid * NC + cid
        cbase = wid * NL
        rbase = sid * rpt

        pltpu.async_copy(x_hbm.at[pl.ds(rbase, rpt)],
                         acc.at[pl.ds(rbase, rpt)], nsem)
        if rem:
            @pl.when(sid == NS - 1)
            def _():
                pltpu.async_copy(x_hbm.at[pl.ds(NS * rpt, rem)],
                                 acc.at[pl.ds(NS * rpt, rem)], nsem)

        def drain(sem, buf_ref):
            pltpu.make_async_copy(x_hbm.at[pl.ds(0, CH)], buf_ref, sem).wait()

        def drain_idx(q):
            pltpu.make_async_copy(eidx_hbm.at[pl.ds(0, 2), pl.ds(0, CH)],
                                  sidx[q], isems[q]).wait()

        def fire_idx(j, q):
            pltpu.async_copy(
                eidx_hbm.at[pl.ds(0, 2), pl.ds((cbase + j) * CH, CH)],
                sidx[q], isems[q])

        def fire_gather(b, q):
            pltpu.async_copy(x_hbm.at[sidx[q].at[0]], rows[b], gsems[b])

        for q in range(QR):
            fire_idx(q, q)
        for b in range(NB):
            drain_idx(b)
            fire_gather(b, b)

        pltpu.make_async_copy(x_hbm.at[pl.ds(0, rpt)],
                              acc.at[pl.ds(0, rpt)], nsem).wait()
        if rem:
            @pl.when(sid == NS - 1)
            def _():
                pltpu.make_async_copy(x_hbm.at[pl.ds(0, rem)],
                                      acc.at[pl.ds(0, rem)], nsem).wait()
        plsc.subcore_barrier()

        def step(i, b, q, do_fidx, do_fg):
            drain(gsems[b], rows[b])
            pltpu.sync_copy(rows[b], acc.at[sidx[q].at[1]], add=True)
            if do_fidx:
                fire_idx(i + QR, q)
            if do_fg:
                drain_idx((q + NB) % QR)
                fire_gather(b, (q + NB) % QR)

        n_grp = (NL - QR) // UN
        n_peeled = n_grp * UN

        def outer(g, carry):
            for u in range(UN):
                step(g * UN + u, u % NB, u % QR, True, True)
            return carry

        lax.fori_loop(0, n_grp, outer, 0)

        for i in range(n_peeled, NL):
            step(i, i % NB, i % QR, i + QR < NL, i + NB < NL)

        if extra:
            @pl.when(wid < extra)
            def _():
                jj = nchk - extra + wid
                pltpu.sync_copy(
                    eidx_hbm.at[pl.ds(0, 2), pl.ds(jj * CH, CH)], sidx[0])
                pltpu.async_copy(x_hbm.at[sidx[0].at[0]], rows[0],
                                 gsems[0]).wait()
                pltpu.sync_copy(rows[0], acc.at[sidx[0].at[1]], add=True)

        plsc.subcore_barrier()

        pltpu.sync_copy(acc.at[pl.ds(rbase, rpt)],
                        out_hbm.at[pl.ds(cid * N + rbase, rpt)])
        if rem:
            @pl.when(sid == NS - 1)
            def _():
                pltpu.sync_copy(acc.at[pl.ds(NS * rpt, rem)],
                                out_hbm.at[pl.ds(cid * N + NS * rpt, rem)])

    return sc_kernel(x, eidx)


def _combine(partials, x):
    N, D = x.shape
    br = 2000
    grid = N // br
    assert br * grid == N

    def body(p0_ref, p1_ref, x_ref, o_ref):
        o_ref[...] = p0_ref[...] + p1_ref[...] - x_ref[...]

    return pl.pallas_call(
        body,
        grid=(grid,),
        in_specs=[
            pl.BlockSpec((br, D), lambda i: (i, 0)),
            pl.BlockSpec((br, D), lambda i: (i + grid, 0)),
            pl.BlockSpec((br, D), lambda i: (i, 0)),
        ],
        out_specs=pl.BlockSpec((br, D), lambda i: (i, 0)),
        out_shape=jax.ShapeDtypeStruct((N, D), jnp.float32),
    )(partials, partials, x)


def kernel(x, edge_index):
    E = edge_index.shape[1]
    assert E % CH == 0
    partials = _sc_partials(x, edge_index)
    return _combine(partials, x)
